# baseline (speedup 1.0000x reference)
import functools

import jax
import jax.numpy as jnp
from jax import lax
from jax.experimental import pallas as pl
from jax.experimental.pallas import tpu as pltpu
from jax.experimental.pallas import tpu_sc as plsc

_LANES = 16
_CORES = 2
_SUBCORES = 16
_TILES = _CORES * _SUBCORES
_GCHUNK = 128


@functools.partial(jax.jit, static_argnames=("n_g",))
def _sc_partials(table, idx, rew, *, n_g):
    per_tile = n_g * _GCHUNK
    mesh = plsc.VectorSubcoreMesh(
        core_axis_name="c", subcore_axis_name="s",
        num_cores=_CORES, num_subcores=_SUBCORES,
    )

    @functools.partial(
        pl.kernel,
        mesh=mesh,
        out_type=jax.ShapeDtypeStruct((_TILES, _LANES), jnp.float32),
        scratch_types=[
            pltpu.VMEM((per_tile,), jnp.int32),
            pltpu.VMEM((per_tile,), jnp.float32),
            pltpu.VMEM((n_g, _GCHUNK), jnp.float32),
            pltpu.VMEM((_LANES,), jnp.float32),
            pltpu.SemaphoreType.DMA,
        ],
    )
    def body(table_hbm, idx_hbm, rew_hbm, out_hbm,
             idx_v, rew_v, vals_v, part_v, sem):
        wid = lax.axis_index("s") * _CORES + lax.axis_index("c")
        base = wid * per_tile

        pltpu.sync_copy(idx_hbm.at[pl.ds(base, per_tile)], idx_v)
        pltpu.sync_copy(rew_hbm.at[pl.ds(base, per_tile)], rew_v)

        descs = [
            pltpu.make_async_copy(
                table_hbm.at[idx_v.at[pl.ds(g * _GCHUNK, _GCHUNK)]],
                vals_v.at[g], sem)
            for g in range(n_g)
        ]
        for d in descs:
            d.start()
        acc = jnp.zeros((_LANES,), jnp.float32)
        for g in range(n_g):
            descs[g].wait()
            for k in range(_GCHUNK // _LANES):
                v = vals_v[g, pl.ds(k * _LANES, _LANES)]
                w = rew_v[pl.ds(g * _GCHUNK + k * _LANES, _LANES)]
                acc = acc + v * w
        part_v[...] = acc
        pltpu.sync_copy(part_v, out_hbm.at[wid])

    return body(table, idx, rew)


def _reduce_body(p_ref, o_ref):
    o_ref[0, 0] = -jnp.sum(p_ref[...])


@jax.jit
def _tc_reduce(partials):
    out = pl.pallas_call(
        _reduce_body,
        out_shape=jax.ShapeDtypeStruct((1, 1), jnp.float32),
        out_specs=pl.BlockSpec(memory_space=pltpu.SMEM),
    )(partials)
    return out[0, 0]


def kernel(seqLogprobs, reward, batchsize_cap, target):
    b, t, vocab = seqLogprobs.shape
    rows = b * t
    n_g = -(-rows // (_TILES * _GCHUNK))
    padded = _TILES * n_g * _GCHUNK

    table = jnp.transpose(seqLogprobs, (1, 2, 0)).reshape(-1)
    r = jnp.arange(rows, dtype=jnp.int32)
    flat_idx = ((r % t) * (vocab * b)
                + target.astype(jnp.int32) * b
                + r // t)
    idx = jnp.pad(flat_idx, (0, padded - rows))
    rew = jnp.pad(reward.astype(jnp.float32), (0, padded - rows))

    partials = _sc_partials(table, idx, rew, n_g=n_g)
    return _tc_reduce(partials)


# speedup vs baseline: 1.0366x; 1.0366x over previous
import functools

import jax
import jax.numpy as jnp
from jax import lax
from jax.experimental import pallas as pl
from jax.experimental.pallas import tpu as pltpu
from jax.experimental.pallas import tpu_sc as plsc

_LANES = 16
_CORES = 1
_SUBCORES = 16
_TILES = _CORES * _SUBCORES
_GCHUNK = 128


@functools.partial(jax.jit, static_argnames=("n_g",))
def _sc_partials(table, idx, rew, *, n_g):
    per_tile = n_g * _GCHUNK
    mesh = plsc.VectorSubcoreMesh(
        core_axis_name="c", subcore_axis_name="s",
        num_cores=_CORES, num_subcores=_SUBCORES,
    )

    @functools.partial(
        pl.kernel,
        mesh=mesh,
        out_type=jax.ShapeDtypeStruct((_TILES, _LANES), jnp.float32),
        scratch_types=[
            pltpu.VMEM((per_tile,), jnp.int32),
            pltpu.VMEM((per_tile,), jnp.float32),
            pltpu.VMEM((n_g, _GCHUNK), jnp.float32),
            pltpu.VMEM((_LANES,), jnp.float32),
            pltpu.SemaphoreType.DMA,
        ],
    )
    def body(table_hbm, idx_hbm, rew_hbm, out_hbm,
             idx_v, rew_v, vals_v, part_v, sem):
        wid = lax.axis_index("s") * _CORES + lax.axis_index("c")
        base = wid * per_tile

        pltpu.sync_copy(idx_hbm.at[pl.ds(base, per_tile)], idx_v)
        pltpu.sync_copy(rew_hbm.at[pl.ds(base, per_tile)], rew_v)

        descs = [
            pltpu.make_async_copy(
                table_hbm.at[idx_v.at[pl.ds(g * _GCHUNK, _GCHUNK)]],
                vals_v.at[g], sem)
            for g in range(n_g)
        ]
        for d in descs:
            d.start()
        acc = jnp.zeros((_LANES,), jnp.float32)
        for g in range(n_g):
            descs[g].wait()
            for k in range(_GCHUNK // _LANES):
                v = vals_v[g, pl.ds(k * _LANES, _LANES)]
                w = rew_v[pl.ds(g * _GCHUNK + k * _LANES, _LANES)]
                acc = acc + v * w
        part_v[...] = acc
        pltpu.sync_copy(part_v, out_hbm.at[wid])

    return body(table, idx, rew)


def _reduce_body(p_ref, o_ref):
    o_ref[0, 0] = -jnp.sum(p_ref[...])


@jax.jit
def _tc_reduce(partials):
    out = pl.pallas_call(
        _reduce_body,
        out_shape=jax.ShapeDtypeStruct((1, 1), jnp.float32),
        out_specs=pl.BlockSpec(memory_space=pltpu.SMEM),
    )(partials)
    return out[0, 0]


def kernel(seqLogprobs, reward, batchsize_cap, target):
    b, t, vocab = seqLogprobs.shape
    rows = b * t
    n_g = -(-rows // (_TILES * _GCHUNK))
    padded = _TILES * n_g * _GCHUNK

    table = jnp.transpose(seqLogprobs, (1, 2, 0)).reshape(-1)
    r = jnp.arange(rows, dtype=jnp.int32)
    flat_idx = ((r % t) * (vocab * b)
                + target.astype(jnp.int32) * b
                + r // t)
    idx = jnp.pad(flat_idx, (0, padded - rows))
    rew = jnp.pad(reward.astype(jnp.float32), (0, padded - rows))

    partials = _sc_partials(table, idx, rew, n_g=n_g)
    return _tc_reduce(partials)


# speedup vs baseline: 1.2822x; 1.2369x over previous
import functools

import jax
import jax.numpy as jnp
from jax import lax
from jax.experimental import pallas as pl
from jax.experimental.pallas import tpu as pltpu
from jax.experimental.pallas import tpu_sc as plsc

_LANES = 16
_CORES = 1
_SUBCORES = 16
_TILES = _CORES * _SUBCORES
_GCHUNK = 128


@functools.partial(jax.jit, static_argnames=("rows",))
def _sc_partials(table, idx, rew, *, rows):
    total_g = rows // _GCHUNK
    n_g = -(-total_g // _TILES)
    per_tile = n_g * _GCHUNK
    full_tiles = total_g // n_g
    rem_g = total_g - full_tiles * n_g

    mesh = plsc.VectorSubcoreMesh(
        core_axis_name="c", subcore_axis_name="s",
        num_cores=_CORES, num_subcores=_SUBCORES,
    )

    @functools.partial(
        pl.kernel,
        mesh=mesh,
        out_type=jax.ShapeDtypeStruct((_TILES, _LANES), jnp.float32),
        scratch_types=[
            pltpu.VMEM((per_tile,), jnp.int32),
            pltpu.VMEM((per_tile,), jnp.float32),
            pltpu.VMEM((n_g, _GCHUNK), jnp.float32),
            pltpu.VMEM((_LANES,), jnp.float32),
            pltpu.SemaphoreType.DMA,
        ],
    )
    def body(table_hbm, idx_hbm, rew_hbm, out_hbm,
             idx_v, rew_v, vals_v, part_v, sem):
        wid = lax.axis_index("s") * _CORES + lax.axis_index("c")
        base = wid * per_tile

        def run_tile(g_count):
            n = g_count * _GCHUNK
            pltpu.sync_copy(idx_hbm.at[pl.ds(base, n)], idx_v.at[pl.ds(0, n)])
            pltpu.sync_copy(rew_hbm.at[pl.ds(base, n)], rew_v.at[pl.ds(0, n)])
            descs = [
                pltpu.make_async_copy(
                    table_hbm.at[idx_v.at[pl.ds(g * _GCHUNK, _GCHUNK)]],
                    vals_v.at[g], sem)
                for g in range(g_count)
            ]
            for d in descs:
                d.start()
            acc = jnp.zeros((_LANES,), jnp.float32)
            for g in range(g_count):
                descs[g].wait()
                for k in range(_GCHUNK // _LANES):
                    v = vals_v[g, pl.ds(k * _LANES, _LANES)]
                    w = rew_v[pl.ds(g * _GCHUNK + k * _LANES, _LANES)]
                    acc = acc + v * w
            part_v[...] = acc
            pltpu.sync_copy(part_v, out_hbm.at[wid])

        @pl.when(wid < full_tiles)
        def _():
            run_tile(n_g)

        if rem_g:
            @pl.when(wid == full_tiles)
            def _():
                run_tile(rem_g)

        empty_from = full_tiles + (1 if rem_g else 0)
        if empty_from < _TILES:
            @pl.when(wid >= empty_from)
            def _():
                part_v[...] = jnp.zeros((_LANES,), jnp.float32)
                pltpu.sync_copy(part_v, out_hbm.at[wid])

    return body(table, idx, rew)


def _reduce_body(p_ref, o_ref):
    o_ref[0, 0] = -jnp.sum(p_ref[...])


@jax.jit
def _tc_reduce(partials):
    out = pl.pallas_call(
        _reduce_body,
        out_shape=jax.ShapeDtypeStruct((1, 1), jnp.float32),
        out_specs=pl.BlockSpec(memory_space=pltpu.SMEM),
    )(partials)
    return out[0, 0]


def kernel(seqLogprobs, reward, batchsize_cap, target):
    b, t, vocab = seqLogprobs.shape
    rows = b * t

    table = jnp.transpose(seqLogprobs, (1, 2, 0)).reshape(-1)
    r = jnp.arange(rows, dtype=jnp.int32)
    flat_idx = ((r % t) * (vocab * b)
                + target.astype(jnp.int32) * b
                + r // t)
    rew = reward.astype(jnp.float32)

    if rows % _GCHUNK:
        pad = _GCHUNK - rows % _GCHUNK
        flat_idx = jnp.pad(flat_idx, (0, pad))
        rew = jnp.pad(rew, (0, pad))

    partials = _sc_partials(table, flat_idx, rew,
                            rows=flat_idx.shape[0])
    return _tc_reduce(partials)


# speedup vs baseline: 1.3041x; 1.0171x over previous
import functools

import jax
import jax.numpy as jnp
from jax import lax
from jax.experimental import pallas as pl
from jax.experimental.pallas import tpu as pltpu
from jax.experimental.pallas import tpu_sc as plsc

_LANES = 16
_CORES = 1
_SUBCORES = 16
_TILES = _CORES * _SUBCORES
_GCHUNK = 128


@functools.partial(jax.jit, static_argnames=("rows",))
def _sc_partials(table, idx, rew, *, rows):
    total_g = rows // _GCHUNK
    n_g = -(-total_g // _TILES)
    per_tile = n_g * _GCHUNK
    full_tiles = total_g // n_g
    rem_g = total_g - full_tiles * n_g

    mesh = plsc.VectorSubcoreMesh(
        core_axis_name="c", subcore_axis_name="s",
        num_cores=_CORES, num_subcores=_SUBCORES,
    )

    @functools.partial(
        pl.kernel,
        mesh=mesh,
        out_type=jax.ShapeDtypeStruct((_TILES, _LANES), jnp.float32),
        scratch_types=[
            pltpu.VMEM((per_tile,), jnp.int32),
            pltpu.VMEM((per_tile,), jnp.float32),
            pltpu.VMEM((n_g, _GCHUNK), jnp.float32),
            pltpu.VMEM((_LANES,), jnp.float32),
            pltpu.SemaphoreType.DMA,
        ],
    )
    def body(table_hbm, idx_hbm, rew_hbm, out_hbm,
             idx_v, rew_v, vals_v, part_v, sem):
        wid = lax.axis_index("s") * _CORES + lax.axis_index("c")
        base = wid * per_tile

        def start_descs(offsets):
            descs = [
                pltpu.make_async_copy(
                    table_hbm.at[idx_v.at[pl.ds(off, _GCHUNK)]],
                    vals_v.at[g], sem)
                for g, off in enumerate(offsets)
            ]
            for d in descs:
                d.start()

        active = full_tiles + (1 if rem_g else 0)

        @pl.when(wid < full_tiles)
        def _():
            n = n_g * _GCHUNK
            pltpu.sync_copy(idx_hbm.at[pl.ds(base, n)], idx_v)
            pltpu.sync_copy(rew_hbm.at[pl.ds(base, n)], rew_v)
            start_descs([g * _GCHUNK for g in range(n_g)])

        if rem_g:
            @pl.when(wid == full_tiles)
            def _():
                n = rem_g * _GCHUNK
                pltpu.sync_copy(idx_hbm.at[pl.ds(base, n)],
                                idx_v.at[pl.ds(0, n)])
                pltpu.sync_copy(rew_hbm.at[pl.ds(base, n)],
                                rew_v.at[pl.ds(0, n)])
                zero = jnp.zeros((_LANES,), jnp.float32)
                for j in range(n, n_g * _GCHUNK, _LANES):
                    rew_v[pl.ds(j, _LANES)] = zero
                start_descs([min(g, rem_g - 1) * _GCHUNK for g in range(n_g)])

        @pl.when(wid < active)
        def _():
            acc = jnp.zeros((_LANES,), jnp.float32)
            for g in range(n_g):
                pltpu.make_async_copy(
                    table_hbm.at[idx_v.at[pl.ds(0, _GCHUNK)]],
                    vals_v.at[g], sem).wait()
                for k in range(_GCHUNK // _LANES):
                    v = vals_v[g, pl.ds(k * _LANES, _LANES)]
                    w = rew_v[pl.ds(g * _GCHUNK + k * _LANES, _LANES)]
                    acc = acc + v * w
            part_v[...] = acc
            pltpu.sync_copy(part_v, out_hbm.at[wid])

        if active < _TILES:
            @pl.when(wid >= active)
            def _():
                part_v[...] = jnp.zeros((_LANES,), jnp.float32)
                pltpu.sync_copy(part_v, out_hbm.at[wid])

    return body(table, idx, rew)


def _reduce_body(p_ref, o_ref):
    o_ref[0, 0] = -jnp.sum(p_ref[...])


@jax.jit
def _tc_reduce(partials):
    out = pl.pallas_call(
        _reduce_body,
        out_shape=jax.ShapeDtypeStruct((1, 1), jnp.float32),
        out_specs=pl.BlockSpec(memory_space=pltpu.SMEM),
    )(partials)
    return out[0, 0]


def kernel(seqLogprobs, reward, batchsize_cap, target):
    b, t, vocab = seqLogprobs.shape
    rows = b * t

    table = jnp.transpose(seqLogprobs, (1, 2, 0)).reshape(-1)
    r = jnp.arange(rows, dtype=jnp.int32)
    flat_idx = ((r % t) * (vocab * b)
                + target.astype(jnp.int32) * b
                + r // t)
    rew = reward.astype(jnp.float32)

    if rows % _GCHUNK:
        pad = _GCHUNK - rows % _GCHUNK
        flat_idx = jnp.pad(flat_idx, (0, pad))
        rew = jnp.pad(rew, (0, pad))

    partials = _sc_partials(table, flat_idx, rew,
                            rows=flat_idx.shape[0])
    return _tc_reduce(partials)
